# baseline (speedup 1.0000x reference)
import functools

import jax
import jax.numpy as jnp
from jax import lax
from jax.experimental import pallas as pl
from jax.experimental.pallas import tpu as pltpu
from jax.experimental.pallas import tpu_sc as plsc

_N_ELEMENTS = 1000000
_D = 16
_B = 16384

_NC = 2
_NS = 16
_NW = _NC * _NS
_BPW = _B // _NW
_HW = _BPW // 2


@functools.partial(
    pl.kernel,
    mesh=plsc.VectorSubcoreMesh(core_axis_name="c", subcore_axis_name="s"),
    out_type=[
        jax.ShapeDtypeStruct((_B, _D), jnp.float32),
        jax.ShapeDtypeStruct((_B, _D), jnp.float32),
    ],
    scratch_types=[
        pltpu.VMEM((_BPW,), jnp.int32),
        pltpu.VMEM((_HW, _D), jnp.float32),
        pltpu.VMEM((_HW, _D), jnp.float32),
        pltpu.SemaphoreType.DMA,
        pltpu.SemaphoreType.DMA,
    ],
)
def _sc_gather(idx_hbm, mean_hbm, rawvar_hbm, out_m, out_v,
               idx_v, buf_m, buf_v, sem_m, sem_v):
    wid = lax.axis_index("s") * _NC + lax.axis_index("c")
    base = wid * _BPW
    pltpu.sync_copy(idx_hbm.at[pl.ds(base, _BPW)], idx_v)

    def fetch(tbl, buf, sem, wave):
        def gbody(g, carry):
            vec = idx_v[pl.ds(wave * _HW + g * 16, 16)]
            for l in range(16):
                pltpu.async_copy(
                    tbl.at[pl.ds(vec[l], 1), :],
                    buf.at[pl.ds(g * 16 + l, 1), :], sem)
            return carry

        lax.fori_loop(0, _HW // 16, gbody, 0)

    def drain(buf, sem):
        pltpu.make_async_copy(mean_hbm.at[pl.ds(0, _HW), :], buf, sem).wait()

    for wave in range(2):
        fetch(mean_hbm, buf_m, sem_m, wave)
        fetch(rawvar_hbm, buf_v, sem_v, wave)
        drain(buf_m, sem_m)
        pltpu.sync_copy(buf_m, out_m.at[pl.ds(base + wave * _HW, _HW)])
        drain(buf_v, sem_v)
        pltpu.sync_copy(buf_v, out_v.at[pl.ds(base + wave * _HW, _HW)])


def _softplus_stack_body(m_ref, v_ref, o_ref):
    o_ref[0] = m_ref[:]
    x = v_ref[:]
    o_ref[1] = jnp.maximum(x, 0.0) + jnp.log1p(jnp.exp(-jnp.abs(x)))


_RB = 1024


def _softplus_stack(ms, vs):
    return pl.pallas_call(
        _softplus_stack_body,
        grid=(_B // _RB,),
        in_specs=[
            pl.BlockSpec((_RB, _D), lambda i: (i, 0)),
            pl.BlockSpec((_RB, _D), lambda i: (i, 0)),
        ],
        out_specs=pl.BlockSpec((2, _RB, _D), lambda i: (0, i, 0)),
        out_shape=jax.ShapeDtypeStruct((2, _B, _D), jnp.float32),
    )(ms, vs)


def kernel(indices, variational_mean, raw_variational_variance):
    idx = indices.astype(jnp.int32)
    ms, vs_raw = _sc_gather(idx, variational_mean, raw_variational_variance)
    return _softplus_stack(ms, vs_raw)
